# baseline (speedup 1.0000x reference)
import functools

import jax
import jax.numpy as jnp
from jax import lax
from jax.experimental import pallas as pl
from jax.experimental.pallas import tpu as pltpu
from jax.experimental.pallas import tpu_sc as plsc

_BATCH = 256
_N_TOK = 77
_PAD_TOK = 80
_D = 768
_LANES = 16
_REGIONS = ((0, 16), (16, 64))


def _sc_embed(tokens_pflat, table, pos, pos_tail):
    info = plsc.get_sparse_core_info()
    num_workers = info.num_cores * info.num_subcores
    b_per_w = _BATCH // num_workers

    mesh = plsc.VectorSubcoreMesh(core_axis_name="c", subcore_axis_name="s")

    @functools.partial(
        pl.kernel,
        mesh=mesh,
        out_type=jax.ShapeDtypeStruct((_BATCH, _N_TOK, _D), jnp.float32),
        scratch_types=[
            pltpu.VMEM((b_per_w * _PAD_TOK,), jnp.int32),
            pltpu.VMEM((_PAD_TOK, _D), jnp.float32),
            pltpu.VMEM((_PAD_TOK, _D), jnp.float32),
        ] + [pltpu.VMEM((n,), jnp.int32) for _, n in _REGIONS]
        + [pltpu.SemaphoreType.DMA] * (2 * len(_REGIONS)),
    )
    def body(
        tok_hbm, tab_hbm, pos_hbm, ptail_hbm, out_hbm,
        idx_v, buf, pos_v, *map_and_sems
    ):
        nr = len(_REGIONS)
        maps = map_and_sems[:nr]
        gsems = map_and_sems[nr:2 * nr]
        ssems = map_and_sems[2 * nr:3 * nr]
        wid = lax.axis_index("s") * info.num_cores + lax.axis_index("c")
        b0 = wid * b_per_w

        lanes = lax.iota(jnp.int32, _LANES)
        fold = lanes - jnp.where(lanes >= 13, 3, 0)
        for r, (o, n) in enumerate(_REGIONS):
            for t in range(n // _LANES):
                base = o + t * _LANES
                vals = fold + 64 if base == 64 else lanes + base
                maps[r][pl.ds(t * _LANES, _LANES)] = vals

        pltpu.sync_copy(pos_hbm.at[pl.ds(0, 72)], pos_v.at[pl.ds(0, 72)])
        pltpu.sync_copy(ptail_hbm, pos_v.at[pl.ds(72, 8)])

        pltpu.sync_copy(
            tok_hbm.at[pl.ds(b0 * _PAD_TOK, b_per_w * _PAD_TOK)], idx_v
        )

        def start_gather(j, r):
            o, n = _REGIONS[r]
            pltpu.async_copy(
                tab_hbm.at[idx_v.at[pl.ds(j * _PAD_TOK + o, n)]],
                buf.at[pl.ds(o, n)],
                gsems[r],
            )

        def wait_gather(r):
            o, n = _REGIONS[r]
            pltpu.make_async_copy(
                tab_hbm.at[idx_v.at[pl.ds(0, n)]], buf.at[pl.ds(o, n)], gsems[r]
            ).wait()

        def start_scatter(b, r):
            o, n = _REGIONS[r]
            pltpu.async_copy(
                buf.at[pl.ds(o, n)], out_hbm.at[b].at[maps[r]], ssems[r]
            )

        def wait_scatter(b, r):
            o, n = _REGIONS[r]
            pltpu.make_async_copy(
                buf.at[pl.ds(o, n)], out_hbm.at[b].at[maps[r]], ssems[r]
            ).wait()

        def add_region(r):
            o, n = _REGIONS[r]

            @plsc.parallel_loop(o, o + n)
            def _(i):
                @plsc.parallel_loop(0, _D // _LANES, unroll=8)
                def _(k):
                    sl = pl.ds(k * _LANES, _LANES)
                    buf[i, sl] = buf[i, sl] + pos_v[i, sl]

        def batch(j, carry):
            b = b0 + j
            for r in range(len(_REGIONS)):
                @pl.when(j > 0)
                def _():
                    wait_scatter(b - 1, r)

                start_gather(j, r)
            for r in range(len(_REGIONS)):
                wait_gather(r)
                add_region(r)
                start_scatter(b, r)
            return carry

        lax.fori_loop(0, b_per_w, batch, 0)
        for r in range(len(_REGIONS)):
            wait_scatter(b0 + b_per_w - 1, r)

    return body(tokens_pflat, table, pos, pos_tail)


def kernel(tokens, token_embedding, positional_embedding):
    tokens_pad = jnp.concatenate(
        [tokens.astype(jnp.int32), tokens[:, 74:77].astype(jnp.int32)], axis=1
    ).reshape(-1)
    tail_rows = jnp.array([72, 73, 74, 75, 76, 74, 75, 76], dtype=jnp.int32)
    pos_tail = jnp.take(positional_embedding, tail_rows, axis=0)
    return _sc_embed(
        tokens_pad, token_embedding, positional_embedding, pos_tail
    )


# speedup vs baseline: 1.0528x; 1.0528x over previous
import functools

import jax
import jax.numpy as jnp
from jax import lax
from jax.experimental import pallas as pl
from jax.experimental.pallas import tpu as pltpu
from jax.experimental.pallas import tpu_sc as plsc

_BATCH = 256
_N_TOK = 77
_PAD_TOK = 80
_D = 768
_LANES = 16
_REGIONS = ((0, 48), (48, 32))


def _sc_embed(tokens_pflat, table, pos, pos_tail):
    info = plsc.get_sparse_core_info()
    num_workers = info.num_cores * info.num_subcores
    b_per_w = _BATCH // num_workers

    mesh = plsc.VectorSubcoreMesh(core_axis_name="c", subcore_axis_name="s")

    @functools.partial(
        pl.kernel,
        mesh=mesh,
        out_type=jax.ShapeDtypeStruct((_BATCH, _N_TOK, _D), jnp.float32),
        scratch_types=[
            pltpu.VMEM((b_per_w * _PAD_TOK,), jnp.int32),
            pltpu.VMEM((_PAD_TOK, _D), jnp.float32),
            pltpu.VMEM((_PAD_TOK, _D), jnp.float32),
        ] + [pltpu.VMEM((n,), jnp.int32) for _, n in _REGIONS]
        + [pltpu.SemaphoreType.DMA] * (2 * len(_REGIONS)),
    )
    def body(
        tok_hbm, tab_hbm, pos_hbm, ptail_hbm, out_hbm,
        idx_v, buf, pos_v, *map_and_sems
    ):
        nr = len(_REGIONS)
        maps = map_and_sems[:nr]
        gsems = map_and_sems[nr:2 * nr]
        ssems = map_and_sems[2 * nr:3 * nr]
        wid = lax.axis_index("s") * info.num_cores + lax.axis_index("c")
        b0 = wid * b_per_w

        lanes = lax.iota(jnp.int32, _LANES)
        fold = lanes - jnp.where(lanes >= 13, 3, 0)
        for r, (o, n) in enumerate(_REGIONS):
            for t in range(n // _LANES):
                base = o + t * _LANES
                vals = fold + 64 if base == 64 else lanes + base
                maps[r][pl.ds(t * _LANES, _LANES)] = vals

        pltpu.sync_copy(pos_hbm.at[pl.ds(0, 72)], pos_v.at[pl.ds(0, 72)])
        pltpu.sync_copy(ptail_hbm, pos_v.at[pl.ds(72, 8)])

        pltpu.sync_copy(
            tok_hbm.at[pl.ds(b0 * _PAD_TOK, b_per_w * _PAD_TOK)], idx_v
        )

        def start_gather(j, r):
            o, n = _REGIONS[r]
            pltpu.async_copy(
                tab_hbm.at[idx_v.at[pl.ds(j * _PAD_TOK + o, n)]],
                buf.at[pl.ds(o, n)],
                gsems[r],
            )

        def wait_gather(r):
            o, n = _REGIONS[r]
            pltpu.make_async_copy(
                tab_hbm.at[idx_v.at[pl.ds(0, n)]], buf.at[pl.ds(o, n)], gsems[r]
            ).wait()

        def start_scatter(b, r):
            o, n = _REGIONS[r]
            pltpu.async_copy(
                buf.at[pl.ds(o, n)], out_hbm.at[b].at[maps[r]], ssems[r]
            )

        def wait_scatter(b, r):
            o, n = _REGIONS[r]
            pltpu.make_async_copy(
                buf.at[pl.ds(o, n)], out_hbm.at[b].at[maps[r]], ssems[r]
            ).wait()

        def add_region(r):
            o, n = _REGIONS[r]

            @plsc.parallel_loop(o, o + n)
            def _(i):
                @plsc.parallel_loop(0, _D // _LANES, unroll=8)
                def _(k):
                    sl = pl.ds(k * _LANES, _LANES)
                    buf[i, sl] = buf[i, sl] + pos_v[i, sl]

        def batch(j, carry):
            b = b0 + j
            for r in range(len(_REGIONS)):
                @pl.when(j > 0)
                def _():
                    wait_scatter(b - 1, r)

                start_gather(j, r)
            for r in range(len(_REGIONS)):
                wait_gather(r)
                add_region(r)
                start_scatter(b, r)
            return carry

        lax.fori_loop(0, b_per_w, batch, 0)
        for r in range(len(_REGIONS)):
            wait_scatter(b0 + b_per_w - 1, r)

    return body(tokens_pflat, table, pos, pos_tail)


def kernel(tokens, token_embedding, positional_embedding):
    tokens_pad = jnp.concatenate(
        [tokens.astype(jnp.int32), tokens[:, 74:77].astype(jnp.int32)], axis=1
    ).reshape(-1)
    tail_rows = jnp.array([72, 73, 74, 75, 76, 74, 75, 76], dtype=jnp.int32)
    pos_tail = jnp.take(positional_embedding, tail_rows, axis=0)
    return _sc_embed(
        tokens_pad, token_embedding, positional_embedding, pos_tail
    )


# speedup vs baseline: 1.0900x; 1.0354x over previous
import functools

import jax
import jax.numpy as jnp
from jax import lax
from jax.experimental import pallas as pl
from jax.experimental.pallas import tpu as pltpu
from jax.experimental.pallas import tpu_sc as plsc

_BATCH = 256
_N_TOK = 77
_PAD_TOK = 80
_D = 768
_LANES = 16
_REGIONS = ((0, 32), (32, 48))


def _sc_embed(tokens_pflat, table, pos, pos_tail):
    info = plsc.get_sparse_core_info()
    num_workers = info.num_cores * info.num_subcores
    b_per_w = _BATCH // num_workers

    mesh = plsc.VectorSubcoreMesh(core_axis_name="c", subcore_axis_name="s")

    @functools.partial(
        pl.kernel,
        mesh=mesh,
        out_type=jax.ShapeDtypeStruct((_BATCH, _N_TOK, _D), jnp.float32),
        scratch_types=[
            pltpu.VMEM((b_per_w * _PAD_TOK,), jnp.int32),
            pltpu.VMEM((_PAD_TOK, _D), jnp.float32),
            pltpu.VMEM((_PAD_TOK, _D), jnp.float32),
        ] + [pltpu.VMEM((n,), jnp.int32) for _, n in _REGIONS]
        + [pltpu.SemaphoreType.DMA] * (2 * len(_REGIONS)),
    )
    def body(
        tok_hbm, tab_hbm, pos_hbm, ptail_hbm, out_hbm,
        idx_v, buf, pos_v, *map_and_sems
    ):
        nr = len(_REGIONS)
        maps = map_and_sems[:nr]
        gsems = map_and_sems[nr:2 * nr]
        ssems = map_and_sems[2 * nr:3 * nr]
        wid = lax.axis_index("s") * info.num_cores + lax.axis_index("c")
        b0 = wid * b_per_w

        lanes = lax.iota(jnp.int32, _LANES)
        fold = lanes - jnp.where(lanes >= 13, 3, 0)
        for r, (o, n) in enumerate(_REGIONS):
            for t in range(n // _LANES):
                base = o + t * _LANES
                vals = fold + 64 if base == 64 else lanes + base
                maps[r][pl.ds(t * _LANES, _LANES)] = vals

        pltpu.sync_copy(pos_hbm.at[pl.ds(0, 72)], pos_v.at[pl.ds(0, 72)])
        pltpu.sync_copy(ptail_hbm, pos_v.at[pl.ds(72, 8)])

        pltpu.sync_copy(
            tok_hbm.at[pl.ds(b0 * _PAD_TOK, b_per_w * _PAD_TOK)], idx_v
        )

        def start_gather(j, r):
            o, n = _REGIONS[r]
            pltpu.async_copy(
                tab_hbm.at[idx_v.at[pl.ds(j * _PAD_TOK + o, n)]],
                buf.at[pl.ds(o, n)],
                gsems[r],
            )

        def wait_gather(r):
            o, n = _REGIONS[r]
            pltpu.make_async_copy(
                tab_hbm.at[idx_v.at[pl.ds(0, n)]], buf.at[pl.ds(o, n)], gsems[r]
            ).wait()

        def start_scatter(b, r):
            o, n = _REGIONS[r]
            pltpu.async_copy(
                buf.at[pl.ds(o, n)], out_hbm.at[b].at[maps[r]], ssems[r]
            )

        def wait_scatter(b, r):
            o, n = _REGIONS[r]
            pltpu.make_async_copy(
                buf.at[pl.ds(o, n)], out_hbm.at[b].at[maps[r]], ssems[r]
            ).wait()

        def add_region(r):
            o, n = _REGIONS[r]

            @plsc.parallel_loop(o, o + n)
            def _(i):
                @plsc.parallel_loop(0, _D // _LANES, unroll=8)
                def _(k):
                    sl = pl.ds(k * _LANES, _LANES)
                    buf[i, sl] = buf[i, sl] + pos_v[i, sl]

        def batch(j, carry):
            b = b0 + j
            for r in range(len(_REGIONS)):
                @pl.when(j > 0)
                def _():
                    wait_scatter(b - 1, r)

                start_gather(j, r)
            for r in range(len(_REGIONS)):
                wait_gather(r)
                add_region(r)
                start_scatter(b, r)
            return carry

        lax.fori_loop(0, b_per_w, batch, 0)
        for r in range(len(_REGIONS)):
            wait_scatter(b0 + b_per_w - 1, r)

    return body(tokens_pflat, table, pos, pos_tail)


def kernel(tokens, token_embedding, positional_embedding):
    tokens_pad = jnp.concatenate(
        [tokens.astype(jnp.int32), tokens[:, 74:77].astype(jnp.int32)], axis=1
    ).reshape(-1)
    tail_rows = jnp.array([72, 73, 74, 75, 76, 74, 75, 76], dtype=jnp.int32)
    pos_tail = jnp.take(positional_embedding, tail_rows, axis=0)
    return _sc_embed(
        tokens_pad, token_embedding, positional_embedding, pos_tail
    )
